# baseline (speedup 1.0000x reference)
import functools

import jax
import jax.numpy as jnp
from jax import lax
from jax.experimental import pallas as pl
from jax.experimental.pallas import tpu as pltpu
from jax.experimental.pallas import tpu_sc as plsc

DIM = 128
CHUNK = 512
IDXC = 64
_NSUB = 2


_NBUF = 2


def _gather_body(row_off, nw, rpt, d,
                 lhs_idx2, rhs_idx2, table_hbm,
                 lhs_out, rhs_out, idx_l, idx_r, rows_v,
                 gsem0, gsem1, wsem0, wsem1):
  info = plsc.get_sparse_core_info()
  wid = lax.axis_index("s") * info.num_cores + lax.axis_index("c")
  base = wid * rpt * IDXC
  pltpu.sync_copy(lhs_idx2.at[pl.ds(row_off + wid * rpt, rpt)], idx_l)
  pltpu.sync_copy(rhs_idx2.at[pl.ds(row_off + wid * rpt, rpt)], idx_r)
  gsems = (gsem0, gsem1)
  wsems = (wsem0, wsem1)
  ntasks = 2 * rpt
  tasks = [(side, j) for side in range(2) for j in range(rpt)]
  idxs = (idx_l, idx_r)
  outs = (lhs_out, rhs_out)
  gh = [None] * ntasks
  wh = [None] * ntasks
  for t in range(ntasks + 1):
    if t < ntasks:
      side, j = tasks[t]
      bu = t % _NBUF
      if t >= _NBUF:
        wh[t - _NBUF].wait()
      gh[t] = pltpu.async_copy(table_hbm.at[idxs[side].at[j]],
                               rows_v.at[bu], gsems[bu])
    if t >= 1:
      side, j = tasks[t - 1]
      bu = (t - 1) % _NBUF
      gh[t - 1].wait()
      wh[t - 1] = pltpu.async_copy(
          rows_v.at[bu],
          outs[side].at[pl.ds(base + j * IDXC, IDXC)], wsems[bu])
  wh[ntasks - 2].wait()
  wh[ntasks - 1].wait()


def _sc_gather_part(lhs_idx2, rhs_idx2, emb, row_off, n_chunks):
  vocab, d = emb.shape
  info = plsc.get_sparse_core_info()
  nw = info.num_cores * info.num_subcores
  bp = n_chunks * CHUNK
  rpt = bp // (nw * IDXC)
  mesh = plsc.VectorSubcoreMesh(core_axis_name="c", subcore_axis_name="s")
  kern = functools.partial(
      pl.kernel,
      mesh=mesh,
      out_type=[
          jax.ShapeDtypeStruct((bp, d), jnp.float32),
          jax.ShapeDtypeStruct((bp, d), jnp.float32),
      ],
      scratch_types=[
          pltpu.VMEM((rpt, IDXC), jnp.int32),
          pltpu.VMEM((rpt, IDXC), jnp.int32),
          pltpu.VMEM((_NBUF, IDXC, d), jnp.float32),
          pltpu.SemaphoreType.DMA,
          pltpu.SemaphoreType.DMA,
          pltpu.SemaphoreType.DMA,
          pltpu.SemaphoreType.DMA,
      ],
  )(functools.partial(_gather_body, row_off, nw, rpt, d))
  return kern(lhs_idx2, rhs_idx2, emb)


_IBUF = 4
_G = 2


def _score_body(g_off, gp, *refs):
  lhs_hbm, rhs_hbm, rel_ref = refs[0], refs[1], refs[2]
  pos_hbm, ln_hbm, rn_hbm = refs[-13], refs[-12], refs[-11]
  (lhs_buf, rhs_buf, ln_buf, rn_buf, pos_buf,
   sem_lhs, sem_rhs, sem_ln, sem_rn, sem_pos) = refs[-10:]
  i = pl.program_id(0)
  slot = lax.rem(i, 2)
  isl = lax.rem(i, _IBUF)

  def _in_copy(hbm, buf, sems, g, sl):
    return pltpu.make_async_copy(hbm.at[pl.ds(g * _G, _G)], buf.at[sl],
                                 sems.at[sl])

  def _out_copy(buf, hbm, sems, sl, g, start):
    rows = CHUNK // _NSUB
    for j in range(_G):
      chunk = (g + g_off) * _G + j
      for k in range(_NSUB):
        cpy = pltpu.make_async_copy(
            buf.at[sl, j, pl.ds(k * rows, rows)],
            hbm.at[chunk, pl.ds(k * rows, rows)],
            sems.at[sl, j, k])
        if start:
          cpy.start()
        else:
          cpy.wait()

  @pl.when(i == 0)
  def _():
    for k in range(min(_IBUF, gp)):
      _in_copy(lhs_hbm, lhs_buf, sem_lhs, k, k).start()
      _in_copy(rhs_hbm, rhs_buf, sem_rhs, k, k).start()

  @pl.when(i >= 2)
  def _():
    _out_copy(ln_buf, ln_hbm, sem_ln, slot, i - 2, False)
    _out_copy(rn_buf, rn_hbm, sem_rn, slot, i - 2, False)

  _in_copy(lhs_hbm, lhs_buf, sem_lhs, i, isl).wait()
  _in_copy(rhs_hbm, rhs_buf, sem_rhs, i, isl).wait()

  r = lax.broadcasted_iota(jnp.int32, (CHUNK, CHUNK), 0)
  c = lax.broadcasted_iota(jnp.int32, (CHUNK, CHUNK), 1)
  eye = r == c
  neg = jnp.float32(-1e9)
  dn = (((1,), (1,)), ((), ()))
  for j in range(_G):
    lhs = lhs_buf[isl, j]
    rhs = rhs_buf[isl, j] * rel_ref[...]
    s = lax.dot_general(rhs, lhs, dn, preferred_element_type=jnp.float32)
    st = lax.dot_general(lhs, rhs, dn, preferred_element_type=jnp.float32)
    pos_buf[pl.ds(i * _G + j, 1), :] = jnp.sum(lhs * rhs, axis=1)[None]
    ln_buf[slot, j] = jnp.where(eye, neg, s)
    rn_buf[slot, j] = jnp.where(eye, neg, st)
  _out_copy(ln_buf, ln_hbm, sem_ln, slot, i, True)
  _out_copy(rn_buf, rn_hbm, sem_rn, slot, i, True)

  @pl.when(i + _IBUF < gp)
  def _():
    _in_copy(lhs_hbm, lhs_buf, sem_lhs, i + _IBUF, isl).start()
    _in_copy(rhs_hbm, rhs_buf, sem_rhs, i + _IBUF, isl).start()

  @pl.when(i == gp - 1)
  def _():
    pcpy = pltpu.make_async_copy(
        pos_buf, pos_hbm.at[pl.ds(g_off * _G, gp * _G)], sem_pos)
    pcpy.start()
    for g in (gp - 2, gp - 1):
      sl = g % 2
      _out_copy(ln_buf, ln_hbm, sem_ln, sl, g, False)
      _out_copy(rn_buf, rn_hbm, sem_rn, sl, g, False)
    pcpy.wait()


def _tc_score_part(lhs_g, rhs_g, rel_vec, c_off, c_total, prev):
  b, d = lhs_g.shape
  cp = b // CHUNK
  gp = cp // _G
  g_off = c_off // _G
  lhs_c = lhs_g.reshape(cp, CHUNK, d)
  rhs_c = rhs_g.reshape(cp, CHUNK, d)
  rel2 = rel_vec.reshape(1, d)
  in_specs = [
      pl.BlockSpec(memory_space=pl.ANY),
      pl.BlockSpec(memory_space=pl.ANY),
      pl.BlockSpec((1, d), lambda i: (0, 0)),
  ]
  args = [lhs_c, rhs_c, rel2]
  aliases = {}
  if prev is not None:
    for k in range(3):
      in_specs.append(pl.BlockSpec(memory_space=pl.ANY))
      args.append(prev[k])
      aliases[3 + k] = k
  return pl.pallas_call(
      functools.partial(_score_body, g_off, gp),
      grid=(gp,),
      in_specs=in_specs,
      out_specs=[
          pl.BlockSpec(memory_space=pl.ANY),
          pl.BlockSpec(memory_space=pl.ANY),
          pl.BlockSpec(memory_space=pl.ANY),
      ],
      out_shape=[
          jax.ShapeDtypeStruct((c_total, CHUNK), jnp.float32),
          jax.ShapeDtypeStruct((c_total, CHUNK, CHUNK), jnp.float32),
          jax.ShapeDtypeStruct((c_total, CHUNK, CHUNK), jnp.float32),
      ],
      scratch_shapes=[
          pltpu.VMEM((_IBUF, _G, CHUNK, DIM), jnp.float32),
          pltpu.VMEM((_IBUF, _G, CHUNK, DIM), jnp.float32),
          pltpu.VMEM((2, _G, CHUNK, CHUNK), jnp.float32),
          pltpu.VMEM((2, _G, CHUNK, CHUNK), jnp.float32),
          pltpu.VMEM((cp, CHUNK), jnp.float32),
          pltpu.SemaphoreType.DMA((_IBUF,)),
          pltpu.SemaphoreType.DMA((_IBUF,)),
          pltpu.SemaphoreType.DMA((2, _G, _NSUB)),
          pltpu.SemaphoreType.DMA((2, _G, _NSUB)),
          pltpu.SemaphoreType.DMA,
      ],
      input_output_aliases=aliases,
  )(*args)


_PARTS = (4, 4, 8, 16)


def kernel(lhs_idx, rhs_idx, emb, rel_vec):
  b = lhs_idx.shape[0]
  c_total = b // CHUNK
  lhs_idx2 = lhs_idx.reshape(b // IDXC, IDXC).astype(jnp.int32)
  rhs_idx2 = rhs_idx.reshape(b // IDXC, IDXC).astype(jnp.int32)
  gathered = []
  row_off = 0
  for n_chunks in _PARTS:
    gathered.append(_sc_gather_part(lhs_idx2, rhs_idx2, emb, row_off,
                                    n_chunks))
    row_off += n_chunks * CHUNK // IDXC
  prev = None
  c_off = 0
  for p, n_chunks in enumerate(_PARTS):
    prev = _tc_score_part(gathered[p][0], gathered[p][1], rel_vec,
                          c_off, c_total, prev)
    c_off += n_chunks
  pos, ln, rn = prev
  return pos, ln, rn
